# baseline (speedup 1.0000x reference)
from functools import partial

import jax
import jax.numpy as jnp
from jax.experimental import pallas as pl
from jax.experimental.pallas import tpu as pltpu
from jax.experimental.pallas import tpu_sc as plsc

_BETA = 0.25
_CHUNK = 1024
_GATHER_WINDOW = 128


def _vq_tc_kernel(z_ref, w_ref, idx_ref, div_ref, nrm_ref, dsc_ref, zb2_ref,
                  acc_ref, *, n_chunks, n_tokens, n_batches, n_codes):
    b = pl.program_id(0)
    c = pl.program_id(1)

    @pl.when((b == 0) & (c == 0))
    def _init():
        acc_ref[...] = jnp.zeros_like(acc_ref)

    @pl.when(b == 0)
    def _norms():
        wc = w_ref[...]
        nrm_ref[c, :] = jnp.sum(wc * wc, axis=1)

    @pl.when(c == 0)
    def _prescale():
        zb = z_ref[0]
        zb2_ref[...] = zb + zb

    scores2 = jax.lax.dot_general(
        w_ref[...], zb2_ref[...], (((1,), (0,)), ((), ())),
        preferred_element_type=jnp.float32)
    dsc_ref[pl.ds(c * _CHUNK, _CHUNK), :] = (
        nrm_ref[c, :].reshape(_CHUNK, 1) - scores2)

    @pl.when(c == n_chunks - 1)
    def _reduce():
        best_idx = jnp.argmin(dsc_ref[...], axis=0).astype(jnp.int32)
        idx_ref[0, 0, :] = best_idx

        tok_row = best_idx.reshape(1, n_tokens)
        tok_col = best_idx.reshape(n_tokens, 1)
        eq = tok_col == tok_row
        srow = jax.lax.broadcasted_iota(jnp.int32, (n_tokens, n_tokens), 0)
        tcol = jax.lax.broadcasted_iota(jnp.int32, (n_tokens, n_tokens), 1)
        dup = jnp.any(eq & (srow < tcol), axis=0)
        dup_vec = jnp.sum(dup.astype(jnp.float32).reshape(-1, 128), axis=0)
        acc_ref[0, :] += jnp.float32(n_tokens / 128.0) - dup_vec

        @pl.when(b == n_batches - 1)
        def _finalize():
            div_ref[0, :] = jnp.full(
                (128,), jnp.sum(acc_ref[0, :]) / (n_tokens * n_batches))


def _nearest_codes(z3, W):
    B, D, T = z3.shape
    N, _ = W.shape
    C = N // _CHUNK
    kern = partial(_vq_tc_kernel, n_chunks=C, n_tokens=T, n_batches=B,
                   n_codes=N)
    return pl.pallas_call(
        kern,
        grid=(B, C),
        in_specs=[
            pl.BlockSpec((1, D, T), lambda b, c: (b, 0, 0)),
            pl.BlockSpec((_CHUNK, D), lambda b, c: (c, 0)),
        ],
        out_specs=[
            pl.BlockSpec((1, 1, T), lambda b, c: (b, 0, 0)),
            pl.BlockSpec((1, 128), lambda b, c: (0, 0)),
        ],
        out_shape=[
            jax.ShapeDtypeStruct((B, 1, T), jnp.int32),
            jax.ShapeDtypeStruct((1, 128), jnp.float32),
        ],
        scratch_shapes=[
            pltpu.VMEM((C, _CHUNK), jnp.float32),
            pltpu.VMEM((N, T), jnp.float32),
            pltpu.VMEM((D, T), jnp.float32),
            pltpu.VMEM((8, 128), jnp.float32),
        ],
    )(z3, W)


def _sc_gather_rows(W, indices):
    _, D = W.shape
    K = indices.shape[1]
    mesh = plsc.VectorSubcoreMesh(core_axis_name="core",
                                  subcore_axis_name="subcore")

    @partial(pl.kernel,
             out_type=jax.ShapeDtypeStruct((K, D), W.dtype),
             mesh=mesh)
    def gather_kernel(x_hbm, i_hbm, o_hbm):
        def body(i_vmem, o_vmem):
            pltpu.sync_copy(x_hbm.at[i_vmem.at[0]], o_vmem)

        pltpu.emit_pipeline(
            body,
            grid=(K // _GATHER_WINDOW,),
            in_specs=[pl.BlockSpec((1, _GATHER_WINDOW),
                                   index_map=lambda i: (0, i))],
            out_specs=[pl.BlockSpec((_GATHER_WINDOW, D),
                                    index_map=lambda i: (i, 0))],
            core_axis_name=("core", "subcore"),
            dimension_semantics=(pltpu.PARALLEL,),
        )(i_hbm, o_hbm)

    return gather_kernel(W, indices)


def _transpose_loss_kernel(zq_ref, z_ref, out_ref, loss_ref, acc_ref, *,
                           n_batches, total_elems):
    b = pl.program_id(0)

    @pl.when(b == 0)
    def _init():
        acc_ref[...] = jnp.zeros_like(acc_ref)

    zq_t = zq_ref[0].T
    out_ref[0] = zq_t
    diff = zq_t - z_ref[0]
    sq = jnp.sum(diff * diff, axis=0)
    acc_ref[0, :] += jnp.sum(sq.reshape(-1, 128), axis=0)

    @pl.when(b == n_batches - 1)
    def _finalize():
        loss_ref[0, :] = jnp.full(
            (128,), (1.0 + _BETA) * jnp.sum(acc_ref[0, :]) / total_elems)


def _transpose_and_loss(zq_rows3, z3):
    B, T, D = zq_rows3.shape
    kern = partial(_transpose_loss_kernel, n_batches=B, total_elems=B * T * D)
    return pl.pallas_call(
        kern,
        grid=(B,),
        in_specs=[
            pl.BlockSpec((1, T, D), lambda b: (b, 0, 0)),
            pl.BlockSpec((1, D, T), lambda b: (b, 0, 0)),
        ],
        out_specs=[
            pl.BlockSpec((1, D, T), lambda b: (b, 0, 0)),
            pl.BlockSpec((1, 128), lambda b: (0, 0)),
        ],
        out_shape=[
            jax.ShapeDtypeStruct((B, D, T), jnp.float32),
            jax.ShapeDtypeStruct((1, 128), jnp.float32),
        ],
        scratch_shapes=[pltpu.VMEM((8, 128), jnp.float32)],
    )(zq_rows3, z3)


def kernel(z, W):
    B, D, H, Wd = z.shape
    T = H * Wd
    z3 = z.reshape(B, D, T)
    idx, div = _nearest_codes(z3, W)
    index = idx.reshape(B, H, Wd)
    diversity = div[0, 0]
    zq_rows = _sc_gather_rows(W, idx.reshape(1, B * T))
    z_q3, loss_vec = _transpose_and_loss(zq_rows.reshape(B, T, D), z3)
    loss = loss_vec[0, 0]
    z_q = z_q3.reshape(B, D, H, Wd)
    return z_q, index, loss, diversity


# speedup vs baseline: 1.5013x; 1.5013x over previous
from functools import partial

import jax
import jax.numpy as jnp
from jax.experimental import pallas as pl
from jax.experimental.pallas import tpu as pltpu
from jax.experimental.pallas import tpu_sc as plsc

_BETA = 0.25
_CHUNK = 1024
_GATHER_WINDOW = 128


def _vq_tc_kernel(z_ref, w_ref, idx_ref, stat_ref, acc_ref, nrm_ref, *,
                  n_chunks, n_tokens, n_batches, total_elems):
    b = pl.program_id(0)

    @pl.when(b == 0)
    def _init():
        acc_ref[...] = jnp.zeros_like(acc_ref)

        def norm_body(c, _):
            wc = w_ref[pl.ds(c * _CHUNK, _CHUNK), :]
            nrm_ref[c, :] = jnp.sum(wc * wc, axis=1)
            return 0

        jax.lax.fori_loop(0, n_chunks, norm_body, 0)

    zb = z_ref[0]
    z2 = jnp.sum(zb * zb, axis=0, keepdims=True)
    zb2 = zb + zb

    def chunk_body(c, carry):
        best_val, best_idx = carry
        wc = w_ref[pl.ds(c * _CHUNK, _CHUNK), :]
        scores2 = jax.lax.dot_general(
            wc, zb2, (((1,), (0,)), ((), ())),
            preferred_element_type=jnp.float32)
        d = nrm_ref[c, :].reshape(_CHUNK, 1) - scores2
        cmin = jnp.min(d, axis=0, keepdims=True)
        cidx = (jnp.argmin(d, axis=0).astype(jnp.int32).reshape(1, n_tokens)
                + c * _CHUNK)
        upd = cmin < best_val
        return (jnp.where(upd, cmin, best_val),
                jnp.where(upd, cidx, best_idx))

    init = (jnp.full((1, n_tokens), jnp.inf, jnp.float32),
            jnp.zeros((1, n_tokens), jnp.int32))
    best_val, best_idx = jax.lax.fori_loop(0, n_chunks, chunk_body, init)
    idx_ref[0, 0, :] = best_idx[0]

    tok_col = best_idx.reshape(n_tokens, 1)
    eq = tok_col == best_idx
    srow = jax.lax.broadcasted_iota(jnp.int32, (n_tokens, n_tokens), 0)
    tcol = jax.lax.broadcasted_iota(jnp.int32, (n_tokens, n_tokens), 1)
    dup = jnp.any(eq & (srow < tcol), axis=0)
    dup_vec = jnp.sum(dup.astype(jnp.float32).reshape(-1, 128), axis=0)

    acc_ref[0, :] += jnp.sum(best_val.reshape(-1, 128), axis=0)
    acc_ref[1, :] += jnp.sum(z2.reshape(-1, 128), axis=0)
    acc_ref[2, :] += jnp.float32(n_tokens / 128.0) - dup_vec

    @pl.when(b == n_batches - 1)
    def _finalize():
        dsum = jnp.sum(acc_ref[0, :])
        zsum = jnp.sum(acc_ref[1, :])
        csum = jnp.sum(acc_ref[2, :])
        loss = (1.0 + _BETA) * (dsum + zsum) / total_elems
        diversity = csum / (n_tokens * n_batches)
        stat_ref[0, :] = jnp.full((128,), loss)
        stat_ref[1, :] = jnp.full((128,), diversity)


def _nearest_codes(z3, W):
    B, D, T = z3.shape
    N, _ = W.shape
    kern = partial(_vq_tc_kernel, n_chunks=N // _CHUNK, n_tokens=T,
                   n_batches=B, total_elems=B * T * D)
    return pl.pallas_call(
        kern,
        grid=(B,),
        in_specs=[
            pl.BlockSpec((1, D, T), lambda b: (b, 0, 0)),
            pl.BlockSpec((N, D), lambda b: (0, 0)),
        ],
        out_specs=[
            pl.BlockSpec((1, 1, T), lambda b: (b, 0, 0)),
            pl.BlockSpec((2, 128), lambda b: (0, 0)),
        ],
        out_shape=[
            jax.ShapeDtypeStruct((B, 1, T), jnp.int32),
            jax.ShapeDtypeStruct((2, 128), jnp.float32),
        ],
        scratch_shapes=[pltpu.VMEM((8, 128), jnp.float32),
                        pltpu.VMEM((N // _CHUNK, _CHUNK), jnp.float32)],
    )(z3, W)


def _sc_gather_rows(W, indices):
    _, D = W.shape
    K = indices.shape[1]
    mesh = plsc.VectorSubcoreMesh(core_axis_name="core",
                                  subcore_axis_name="subcore")

    @partial(pl.kernel,
             out_type=jax.ShapeDtypeStruct((K, D), W.dtype),
             mesh=mesh)
    def gather_kernel(x_hbm, i_hbm, o_hbm):
        def body(i_vmem, o_vmem):
            pltpu.sync_copy(x_hbm.at[i_vmem.at[0]], o_vmem)

        pltpu.emit_pipeline(
            body,
            grid=(K // _GATHER_WINDOW,),
            in_specs=[pl.BlockSpec((1, _GATHER_WINDOW),
                                   index_map=lambda i: (0, i))],
            out_specs=[pl.BlockSpec((_GATHER_WINDOW, D),
                                    index_map=lambda i: (i, 0))],
            core_axis_name=("core", "subcore"),
            dimension_semantics=(pltpu.PARALLEL,),
        )(i_hbm, o_hbm)

    return gather_kernel(W, indices)


def kernel(z, W):
    B, D, H, Wd = z.shape
    T = H * Wd
    z3 = z.reshape(B, D, T)
    idx, stats = _nearest_codes(z3, W)
    index = idx.reshape(B, H, Wd)
    loss = stats[0, 0]
    diversity = stats[1, 0]
    zq_rows = _sc_gather_rows(W, idx.reshape(1, B * T))
    z_q = jnp.moveaxis(zq_rows.reshape(B, H, Wd, D), -1, 1)
    return z_q, index, loss, diversity


# speedup vs baseline: 1.6576x; 1.1041x over previous
from functools import partial

import jax
import jax.numpy as jnp
from jax.experimental import pallas as pl
from jax.experimental.pallas import tpu as pltpu
from jax.experimental.pallas import tpu_sc as plsc

_BETA = 0.25
_CHUNK = 2048
_GATHER_WINDOW = 128


def _vq_tc_kernel(z_ref, w_ref, idx_ref, stat_ref, acc_ref, nrm_ref, *,
                  n_chunks, n_tokens, n_batches, total_elems):
    b = pl.program_id(0)

    @pl.when(b == 0)
    def _init():
        acc_ref[...] = jnp.zeros_like(acc_ref)

        def norm_body(c, _):
            wc = w_ref[pl.ds(c * _CHUNK, _CHUNK), :]
            nrm_ref[c, :] = jnp.sum(wc * wc, axis=1)
            return 0

        jax.lax.fori_loop(0, n_chunks, norm_body, 0)

    zb = z_ref[0]
    z2 = jnp.sum(zb * zb, axis=0, keepdims=True)
    zb2 = zb + zb

    def chunk_body(c, carry):
        best_val, best_idx = carry
        wc = w_ref[pl.ds(c * _CHUNK, _CHUNK), :]
        scores2 = jax.lax.dot_general(
            wc, zb2, (((1,), (0,)), ((), ())),
            preferred_element_type=jnp.float32)
        d = nrm_ref[c, :].reshape(_CHUNK, 1) - scores2
        cmin = jnp.min(d, axis=0, keepdims=True)
        cidx = (jnp.argmin(d, axis=0).astype(jnp.int32).reshape(1, n_tokens)
                + c * _CHUNK)
        upd = cmin < best_val
        return (jnp.where(upd, cmin, best_val),
                jnp.where(upd, cidx, best_idx))

    init = (jnp.full((1, n_tokens), jnp.inf, jnp.float32),
            jnp.zeros((1, n_tokens), jnp.int32))
    best_val, best_idx = jax.lax.fori_loop(0, n_chunks, chunk_body, init)
    idx_ref[0, 0, :] = best_idx[0]

    tok_col = best_idx.reshape(n_tokens, 1)
    eq = tok_col == best_idx
    srow = jax.lax.broadcasted_iota(jnp.int32, (n_tokens, n_tokens), 0)
    tcol = jax.lax.broadcasted_iota(jnp.int32, (n_tokens, n_tokens), 1)
    dup = jnp.any(eq & (srow < tcol), axis=0)
    dup_vec = jnp.sum(dup.astype(jnp.float32).reshape(-1, 128), axis=0)

    acc_ref[0, :] += jnp.sum(best_val.reshape(-1, 128), axis=0)
    acc_ref[1, :] += jnp.sum(z2.reshape(-1, 128), axis=0)
    acc_ref[2, :] += jnp.float32(n_tokens / 128.0) - dup_vec

    @pl.when(b == n_batches - 1)
    def _finalize():
        dsum = jnp.sum(acc_ref[0, :])
        zsum = jnp.sum(acc_ref[1, :])
        csum = jnp.sum(acc_ref[2, :])
        loss = (1.0 + _BETA) * (dsum + zsum) / total_elems
        diversity = csum / (n_tokens * n_batches)
        stat_ref[0, :] = jnp.full((128,), loss)
        stat_ref[1, :] = jnp.full((128,), diversity)


def _nearest_codes(z3, W):
    B, D, T = z3.shape
    N, _ = W.shape
    kern = partial(_vq_tc_kernel, n_chunks=N // _CHUNK, n_tokens=T,
                   n_batches=B, total_elems=B * T * D)
    return pl.pallas_call(
        kern,
        grid=(B,),
        in_specs=[
            pl.BlockSpec((1, D, T), lambda b: (b, 0, 0)),
            pl.BlockSpec((N, D), lambda b: (0, 0)),
        ],
        out_specs=[
            pl.BlockSpec((1, 1, T), lambda b: (b, 0, 0)),
            pl.BlockSpec((2, 128), lambda b: (0, 0)),
        ],
        out_shape=[
            jax.ShapeDtypeStruct((B, 1, T), jnp.int32),
            jax.ShapeDtypeStruct((2, 128), jnp.float32),
        ],
        scratch_shapes=[pltpu.VMEM((8, 128), jnp.float32),
                        pltpu.VMEM((N // _CHUNK, _CHUNK), jnp.float32)],
    )(z3, W)


def _sc_gather_rows(W, indices):
    _, D = W.shape
    K = indices.shape[1]
    mesh = plsc.VectorSubcoreMesh(core_axis_name="core",
                                  subcore_axis_name="subcore")

    @partial(pl.kernel,
             out_type=jax.ShapeDtypeStruct((K, D), W.dtype),
             mesh=mesh)
    def gather_kernel(x_hbm, i_hbm, o_hbm):
        def body(i_vmem, o_vmem):
            pltpu.sync_copy(x_hbm.at[i_vmem.at[0]], o_vmem)

        pltpu.emit_pipeline(
            body,
            grid=(K // _GATHER_WINDOW,),
            in_specs=[pl.BlockSpec((1, _GATHER_WINDOW),
                                   index_map=lambda i: (0, i))],
            out_specs=[pl.BlockSpec((_GATHER_WINDOW, D),
                                    index_map=lambda i: (i, 0))],
            core_axis_name=("core", "subcore"),
            dimension_semantics=(pltpu.PARALLEL,),
        )(i_hbm, o_hbm)

    return gather_kernel(W, indices)


def kernel(z, W):
    B, D, H, Wd = z.shape
    T = H * Wd
    z3 = z.reshape(B, D, T)
    idx, stats = _nearest_codes(z3, W)
    index = idx.reshape(B, H, Wd)
    loss = stats[0, 0]
    diversity = stats[1, 0]
    zq_rows = _sc_gather_rows(W, idx.reshape(1, B * T))
    z_q = jnp.moveaxis(zq_rows.reshape(B, H, Wd, D), -1, 1)
    return z_q, index, loss, diversity


# speedup vs baseline: 1.7422x; 1.0510x over previous
from functools import partial

import jax
import jax.numpy as jnp
from jax.experimental import pallas as pl
from jax.experimental.pallas import tpu as pltpu
from jax.experimental.pallas import tpu_sc as plsc

_BETA = 0.25
_CHUNK = 4096
_GATHER_WINDOW = 128


def _vq_tc_kernel(z_ref, w_ref, idx_ref, stat_ref, acc_ref, nrm_ref, *,
                  n_chunks, n_tokens, n_batches, total_elems):
    b = pl.program_id(0)

    @pl.when(b == 0)
    def _init():
        acc_ref[...] = jnp.zeros_like(acc_ref)

        def norm_body(c, _):
            wc = w_ref[pl.ds(c * _CHUNK, _CHUNK), :]
            nrm_ref[c, :] = jnp.sum(wc * wc, axis=1)
            return 0

        jax.lax.fori_loop(0, n_chunks, norm_body, 0)

    zb = z_ref[0]
    z2 = jnp.sum(zb * zb, axis=0, keepdims=True)
    zb2 = zb + zb

    def chunk_body(c, carry):
        best_val, best_idx = carry
        wc = w_ref[pl.ds(c * _CHUNK, _CHUNK), :]
        scores2 = jax.lax.dot_general(
            wc, zb2, (((1,), (0,)), ((), ())),
            preferred_element_type=jnp.float32)
        d = nrm_ref[c, :].reshape(_CHUNK, 1) - scores2
        cmin = jnp.min(d, axis=0, keepdims=True)
        cidx = (jnp.argmin(d, axis=0).astype(jnp.int32).reshape(1, n_tokens)
                + c * _CHUNK)
        upd = cmin < best_val
        return (jnp.where(upd, cmin, best_val),
                jnp.where(upd, cidx, best_idx))

    init = (jnp.full((1, n_tokens), jnp.inf, jnp.float32),
            jnp.zeros((1, n_tokens), jnp.int32))
    best_val, best_idx = jax.lax.fori_loop(0, n_chunks, chunk_body, init)
    idx_ref[0, 0, :] = best_idx[0]

    tok_col = best_idx.reshape(n_tokens, 1)
    eq = tok_col == best_idx
    srow = jax.lax.broadcasted_iota(jnp.int32, (n_tokens, n_tokens), 0)
    tcol = jax.lax.broadcasted_iota(jnp.int32, (n_tokens, n_tokens), 1)
    dup = jnp.any(eq & (srow < tcol), axis=0)
    dup_vec = jnp.sum(dup.astype(jnp.float32).reshape(-1, 128), axis=0)

    acc_ref[0, :] += jnp.sum(best_val.reshape(-1, 128), axis=0)
    acc_ref[1, :] += jnp.sum(z2.reshape(-1, 128), axis=0)
    acc_ref[2, :] += jnp.float32(n_tokens / 128.0) - dup_vec

    @pl.when(b == n_batches - 1)
    def _finalize():
        dsum = jnp.sum(acc_ref[0, :])
        zsum = jnp.sum(acc_ref[1, :])
        csum = jnp.sum(acc_ref[2, :])
        loss = (1.0 + _BETA) * (dsum + zsum) / total_elems
        diversity = csum / (n_tokens * n_batches)
        stat_ref[0, :] = jnp.full((128,), loss)
        stat_ref[1, :] = jnp.full((128,), diversity)


def _nearest_codes(z3, W):
    B, D, T = z3.shape
    N, _ = W.shape
    kern = partial(_vq_tc_kernel, n_chunks=N // _CHUNK, n_tokens=T,
                   n_batches=B, total_elems=B * T * D)
    return pl.pallas_call(
        kern,
        grid=(B,),
        in_specs=[
            pl.BlockSpec((1, D, T), lambda b: (b, 0, 0)),
            pl.BlockSpec((N, D), lambda b: (0, 0)),
        ],
        out_specs=[
            pl.BlockSpec((1, 1, T), lambda b: (b, 0, 0)),
            pl.BlockSpec((2, 128), lambda b: (0, 0)),
        ],
        out_shape=[
            jax.ShapeDtypeStruct((B, 1, T), jnp.int32),
            jax.ShapeDtypeStruct((2, 128), jnp.float32),
        ],
        scratch_shapes=[pltpu.VMEM((8, 128), jnp.float32),
                        pltpu.VMEM((N // _CHUNK, _CHUNK), jnp.float32)],
    )(z3, W)


def _sc_gather_rows(W, indices):
    _, D = W.shape
    K = indices.shape[1]
    mesh = plsc.VectorSubcoreMesh(core_axis_name="core",
                                  subcore_axis_name="subcore")

    @partial(pl.kernel,
             out_type=jax.ShapeDtypeStruct((K, D), W.dtype),
             mesh=mesh)
    def gather_kernel(x_hbm, i_hbm, o_hbm):
        def body(i_vmem, o_vmem):
            pltpu.sync_copy(x_hbm.at[i_vmem.at[0]], o_vmem)

        pltpu.emit_pipeline(
            body,
            grid=(K // _GATHER_WINDOW,),
            in_specs=[pl.BlockSpec((1, _GATHER_WINDOW),
                                   index_map=lambda i: (0, i))],
            out_specs=[pl.BlockSpec((_GATHER_WINDOW, D),
                                    index_map=lambda i: (i, 0))],
            core_axis_name=("core", "subcore"),
            dimension_semantics=(pltpu.PARALLEL,),
        )(i_hbm, o_hbm)

    return gather_kernel(W, indices)


def kernel(z, W):
    B, D, H, Wd = z.shape
    T = H * Wd
    z3 = z.reshape(B, D, T)
    idx, stats = _nearest_codes(z3, W)
    index = idx.reshape(B, H, Wd)
    loss = stats[0, 0]
    diversity = stats[1, 0]
    zq_rows = _sc_gather_rows(W, idx.reshape(1, B * T))
    z_q = jnp.moveaxis(zq_rows.reshape(B, H, Wd, D), -1, 1)
    return z_q, index, loss, diversity


# speedup vs baseline: 1.8357x; 1.0537x over previous
from functools import partial

import jax
import jax.numpy as jnp
from jax.experimental import pallas as pl
from jax.experimental.pallas import tpu as pltpu
from jax.experimental.pallas import tpu_sc as plsc

_BETA = 0.25
_CHUNK = 8192
_GATHER_WINDOW = 128


def _vq_tc_kernel(z_ref, w_ref, idx_ref, stat_ref, acc_ref, nrm_ref, *,
                  n_chunks, n_tokens, n_batches, total_elems):
    b = pl.program_id(0)

    @pl.when(b == 0)
    def _init():
        acc_ref[...] = jnp.zeros_like(acc_ref)

        def norm_body(c, _):
            wc = w_ref[pl.ds(c * _CHUNK, _CHUNK), :]
            nrm_ref[c, :] = jnp.sum(wc * wc, axis=1)
            return 0

        jax.lax.fori_loop(0, n_chunks, norm_body, 0)

    zb = z_ref[0]
    z2 = jnp.sum(zb * zb, axis=0, keepdims=True)
    zb2 = zb + zb

    def chunk_body(c, carry):
        best_val, best_idx = carry
        wc = w_ref[pl.ds(c * _CHUNK, _CHUNK), :]
        scores2 = jax.lax.dot_general(
            wc, zb2, (((1,), (0,)), ((), ())),
            preferred_element_type=jnp.float32)
        d = nrm_ref[c, :].reshape(_CHUNK, 1) - scores2
        cmin = jnp.min(d, axis=0, keepdims=True)
        cidx = (jnp.argmin(d, axis=0).astype(jnp.int32).reshape(1, n_tokens)
                + c * _CHUNK)
        upd = cmin < best_val
        return (jnp.where(upd, cmin, best_val),
                jnp.where(upd, cidx, best_idx))

    init = (jnp.full((1, n_tokens), jnp.inf, jnp.float32),
            jnp.zeros((1, n_tokens), jnp.int32))
    best_val, best_idx = jax.lax.fori_loop(0, n_chunks, chunk_body, init)
    idx_ref[0, 0, :] = best_idx[0]

    tok_col = best_idx.reshape(n_tokens, 1)
    eq = tok_col == best_idx
    srow = jax.lax.broadcasted_iota(jnp.int32, (n_tokens, n_tokens), 0)
    tcol = jax.lax.broadcasted_iota(jnp.int32, (n_tokens, n_tokens), 1)
    dup = jnp.any(eq & (srow < tcol), axis=0)
    dup_vec = jnp.sum(dup.astype(jnp.float32).reshape(-1, 128), axis=0)

    acc_ref[0, :] += jnp.sum(best_val.reshape(-1, 128), axis=0)
    acc_ref[1, :] += jnp.sum(z2.reshape(-1, 128), axis=0)
    acc_ref[2, :] += jnp.float32(n_tokens / 128.0) - dup_vec

    @pl.when(b == n_batches - 1)
    def _finalize():
        dsum = jnp.sum(acc_ref[0, :])
        zsum = jnp.sum(acc_ref[1, :])
        csum = jnp.sum(acc_ref[2, :])
        loss = (1.0 + _BETA) * (dsum + zsum) / total_elems
        diversity = csum / (n_tokens * n_batches)
        stat_ref[0, :] = jnp.full((128,), loss)
        stat_ref[1, :] = jnp.full((128,), diversity)


def _nearest_codes(z3, W):
    B, D, T = z3.shape
    N, _ = W.shape
    kern = partial(_vq_tc_kernel, n_chunks=N // _CHUNK, n_tokens=T,
                   n_batches=B, total_elems=B * T * D)
    return pl.pallas_call(
        kern,
        grid=(B,),
        in_specs=[
            pl.BlockSpec((1, D, T), lambda b: (b, 0, 0)),
            pl.BlockSpec((N, D), lambda b: (0, 0)),
        ],
        out_specs=[
            pl.BlockSpec((1, 1, T), lambda b: (b, 0, 0)),
            pl.BlockSpec((2, 128), lambda b: (0, 0)),
        ],
        out_shape=[
            jax.ShapeDtypeStruct((B, 1, T), jnp.int32),
            jax.ShapeDtypeStruct((2, 128), jnp.float32),
        ],
        scratch_shapes=[pltpu.VMEM((8, 128), jnp.float32),
                        pltpu.VMEM((N // _CHUNK, _CHUNK), jnp.float32)],
    )(z3, W)


def _sc_gather_rows(W, indices):
    _, D = W.shape
    K = indices.shape[1]
    mesh = plsc.VectorSubcoreMesh(core_axis_name="core",
                                  subcore_axis_name="subcore")

    @partial(pl.kernel,
             out_type=jax.ShapeDtypeStruct((K, D), W.dtype),
             mesh=mesh)
    def gather_kernel(x_hbm, i_hbm, o_hbm):
        def body(i_vmem, o_vmem):
            pltpu.sync_copy(x_hbm.at[i_vmem.at[0]], o_vmem)

        pltpu.emit_pipeline(
            body,
            grid=(K // _GATHER_WINDOW,),
            in_specs=[pl.BlockSpec((1, _GATHER_WINDOW),
                                   index_map=lambda i: (0, i))],
            out_specs=[pl.BlockSpec((_GATHER_WINDOW, D),
                                    index_map=lambda i: (i, 0))],
            core_axis_name=("core", "subcore"),
            dimension_semantics=(pltpu.PARALLEL,),
        )(i_hbm, o_hbm)

    return gather_kernel(W, indices)


def kernel(z, W):
    B, D, H, Wd = z.shape
    T = H * Wd
    z3 = z.reshape(B, D, T)
    idx, stats = _nearest_codes(z3, W)
    index = idx.reshape(B, H, Wd)
    loss = stats[0, 0]
    diversity = stats[1, 0]
    zq_rows = _sc_gather_rows(W, idx.reshape(1, B * T))
    z_q = jnp.moveaxis(zq_rows.reshape(B, H, Wd, D), -1, 1)
    return z_q, index, loss, diversity
